# baseline (speedup 1.0000x reference)
import functools

import jax
import jax.numpy as jnp
from jax import lax
from jax.experimental import pallas as pl
from jax.experimental.pallas import tpu as pltpu
from jax.experimental.pallas import tpu_sc as plsc


def _make_sc_reverse(B, Z):
    info = plsc.get_sparse_core_info()
    NC, NS, L = info.num_cores, info.num_subcores, info.num_lanes
    NW = NC * NS
    rows_per_w = B // NW
    R = 4
    nblocks = rows_per_w // R
    G = Z // L
    H = G // 2
    NBUF = 4

    mesh = plsc.VectorSubcoreMesh(core_axis_name="c", subcore_axis_name="s")

    def _reverse_inplace(buf):
        @plsc.parallel_loop(0, R * H, unroll=8)
        def _(k):
            r = lax.shift_right_logical(k, 7)
            jj = lax.bitwise_and(k, H - 1)
            ja = jj * L
            jb = (G - 1 - jj) * L
            va = buf[r, pl.ds(ja, L)]
            vb = buf[r, pl.ds(jb, L)]
            buf[r, pl.ds(jb, L)] = jnp.flip(va)
            buf[r, pl.ds(ja, L)] = jnp.flip(vb)

    @functools.partial(
        pl.kernel,
        mesh=mesh,
        out_type=jax.ShapeDtypeStruct((B, Z), jnp.float32),
        scratch_types=[
            pltpu.VMEM((R, Z), jnp.float32),
            pltpu.VMEM((R, Z), jnp.float32),
            pltpu.VMEM((R, Z), jnp.float32),
            pltpu.VMEM((R, Z), jnp.float32),
            pltpu.SemaphoreType.DMA,
            pltpu.SemaphoreType.DMA,
            pltpu.SemaphoreType.DMA,
            pltpu.SemaphoreType.DMA,
            pltpu.SemaphoreType.DMA,
            pltpu.SemaphoreType.DMA,
            pltpu.SemaphoreType.DMA,
            pltpu.SemaphoreType.DMA,
        ],
    )
    def k(z_hbm, out_hbm, b0, b1, b2, b3, is0, is1, is2, is3,
          os0, os1, os2, os3):
        wid = lax.axis_index("s") * NC + lax.axis_index("c")
        base = wid * rows_per_w
        bufs = (b0, b1, b2, b3)
        isems = (is0, is1, is2, is3)
        osems = (os0, os1, os2, os3)

        def src_at(b):
            return z_hbm.at[pl.ds(base + b * R, R)]

        def dst_at(b):
            return out_hbm.at[pl.ds(base + b * R, R)]

        pltpu.async_copy(src_at(0), bufs[0], isems[0])
        pltpu.async_copy(src_at(1), bufs[1], isems[1])

        def body(i, carry):
            for p in range(NBUF):
                b = i * NBUF + p
                pltpu.make_async_copy(src_at(b), bufs[p], isems[p]).wait()
                _reverse_inplace(bufs[p])
                pltpu.async_copy(bufs[p], dst_at(b), osems[p])

                pn = (p + 2) % NBUF

                @pl.when(b + 2 < nblocks)
                def _():
                    @pl.when(b >= 2)
                    def _():
                        pltpu.make_async_copy(
                            bufs[pn], dst_at(b - 2), osems[pn]
                        ).wait()

                    pltpu.async_copy(src_at(b + 2), bufs[pn], isems[pn])

            return carry

        lax.fori_loop(0, nblocks // NBUF, body, 0)
        for b in range(nblocks - nblocks % NBUF, nblocks):
            p = b % NBUF
            pltpu.make_async_copy(src_at(b), bufs[p], isems[p]).wait()
            _reverse_inplace(bufs[p])
            pltpu.async_copy(bufs[p], dst_at(b), osems[p])
        for b in range(nblocks - NBUF, nblocks):
            pltpu.make_async_copy(bufs[b % NBUF], dst_at(b),
                                  osems[b % NBUF]).wait()

    return k


def kernel(z):
    B, Z = z.shape
    return _make_sc_reverse(B, Z)(z)
